# initial kernel scaffold (unmeasured)
import jax
import jax.numpy as jnp
from jax import lax
from jax.experimental import pallas as pl
from jax.experimental.pallas import tpu as pltpu

N_DEV = 32


def _gelu(y):
    c = 0.7978845608028654
    return 0.5 * y * (1.0 + jnp.tanh(c * (y + 0.044715 * y * y * y)))


def kernel(x, w_mat):
    m, k_loc = x.shape
    _, n = w_mat.shape
    chunk = m // N_DEV

    def body(x_ref, w_ref, out_ref, rs_buf, ag_buf,
             rs_send_sems, rs_recv_sems, ag_send_sems, ag_recv_sems):
        d = lax.axis_index("i")
        right = lax.rem(d + 1, N_DEV)

        w = w_ref[:, :]

        def pchunk(c):
            return jnp.dot(
                x_ref[pl.ds(c * chunk, chunk), :], w,
                preferred_element_type=jnp.float32,
            )

        for h in range(N_DEV - 1):
            if h == 0:
                c_send = lax.rem(d + (N_DEV - 1), N_DEV)
                rs_buf[N_DEV - 1, :, :] = pchunk(c_send)
                src = rs_buf.at[N_DEV - 1]
            else:
                src = rs_buf.at[h - 1]
            rdma = pltpu.make_async_remote_copy(
                src_ref=src,
                dst_ref=rs_buf.at[h],
                send_sem=rs_send_sems.at[h],
                recv_sem=rs_recv_sems.at[h],
                device_id=(right,),
                device_id_type=pl.DeviceIdType.MESH,
            )
            rdma.start()
            rdma.wait()
            c_recv = lax.rem(d + (2 * N_DEV - 2 - h), N_DEV)
            rs_buf[h, :, :] = rs_buf[h, :, :] + pchunk(c_recv)

        out_ref[pl.ds(d * chunk, chunk), :] = rs_buf[N_DEV - 2, :, :]

        for g in range(N_DEV - 1):
            src = rs_buf.at[N_DEV - 2] if g == 0 else ag_buf.at[g - 1]
            rdma = pltpu.make_async_remote_copy(
                src_ref=src,
                dst_ref=ag_buf.at[g],
                send_sem=ag_send_sems.at[g],
                recv_sem=ag_recv_sems.at[g],
                device_id=(right,),
                device_id_type=pl.DeviceIdType.MESH,
            )
            rdma.start()
            rdma.wait()
            c_recv = lax.rem(d + (2 * N_DEV - 1 - g), N_DEV)
            out_ref[pl.ds(c_recv * chunk, chunk), :] = ag_buf[g, :, :]

        out_ref[:, :] = _gelu(out_ref[:, :])

    return pl.pallas_call(
        body,
        out_shape=jax.ShapeDtypeStruct((m, n), jnp.float32),
        in_specs=[
            pl.BlockSpec(memory_space=pltpu.VMEM),
            pl.BlockSpec(memory_space=pltpu.VMEM),
        ],
        out_specs=pl.BlockSpec(memory_space=pltpu.VMEM),
        scratch_shapes=[
            pltpu.VMEM((N_DEV, chunk, n), jnp.float32),
            pltpu.VMEM((N_DEV - 1, chunk, n), jnp.float32),
            pltpu.SemaphoreType.DMA((N_DEV - 1,)),
            pltpu.SemaphoreType.DMA((N_DEV - 1,)),
            pltpu.SemaphoreType.DMA((N_DEV - 1,)),
            pltpu.SemaphoreType.DMA((N_DEV - 1,)),
        ],
        compiler_params=pltpu.CompilerParams(
            vmem_limit_bytes=60 * 1024 * 1024,
        ),
    )(x, w_mat)


# baseline (device time: 516291 ns/iter reference)
import jax
import jax.numpy as jnp
from jax import lax
from jax.experimental import pallas as pl
from jax.experimental.pallas import tpu as pltpu

N_DEV = 32


def _gelu(y):
    c = 0.7978845608028654
    return 0.5 * y * (1.0 + jnp.tanh(c * (y + 0.044715 * y * y * y)))


def kernel(x, w_mat):
    m, k_loc = x.shape
    _, n = w_mat.shape
    chunk = m // N_DEV

    def body(x_ref, w_ref, out_ref, rs_buf, ag_buf,
             rs_send_sems, rs_recv_sems, ag_send_sems, ag_recv_sems):
        d = lax.axis_index("i")
        right = lax.rem(d + 1, N_DEV)

        w = w_ref[:, :]

        def pchunk(c):
            return jnp.dot(
                x_ref[pl.ds(c * chunk, chunk), :], w,
                preferred_element_type=jnp.float32,
            )

        for h in range(N_DEV - 1):
            if h == 0:
                c_send = lax.rem(d + (N_DEV - 1), N_DEV)
                rs_buf[N_DEV - 1, :, :] = pchunk(c_send)
                src = rs_buf.at[N_DEV - 1]
            else:
                src = rs_buf.at[h - 1]
            rdma = pltpu.make_async_remote_copy(
                src_ref=src,
                dst_ref=rs_buf.at[h],
                send_sem=rs_send_sems.at[h],
                recv_sem=rs_recv_sems.at[h],
                device_id=(right,),
                device_id_type=pl.DeviceIdType.MESH,
            )
            rdma.start()
            rdma.wait()
            c_recv = lax.rem(d + (2 * N_DEV - 2 - h), N_DEV)
            rs_buf[h, :, :] = rs_buf[h, :, :] + pchunk(c_recv)

        out_ref[pl.ds(d * chunk, chunk), :] = rs_buf[N_DEV - 2, :, :]

        for g in range(N_DEV - 1):
            src = rs_buf.at[N_DEV - 2] if g == 0 else ag_buf.at[g - 1]
            rdma = pltpu.make_async_remote_copy(
                src_ref=src,
                dst_ref=ag_buf.at[g],
                send_sem=ag_send_sems.at[g],
                recv_sem=ag_recv_sems.at[g],
                device_id=(right,),
                device_id_type=pl.DeviceIdType.MESH,
            )
            rdma.start()
            rdma.wait()
            c_recv = lax.rem(d + (2 * N_DEV - 1 - g), N_DEV)
            out_ref[pl.ds(c_recv * chunk, chunk), :] = ag_buf[g, :, :]

        for c in range(N_DEV):
            sl = pl.ds(c * chunk, chunk)
            out_ref[sl, :] = _gelu(out_ref[sl, :])

    return pl.pallas_call(
        body,
        out_shape=jax.ShapeDtypeStruct((m, n), jnp.float32),
        in_specs=[
            pl.BlockSpec(memory_space=pltpu.VMEM),
            pl.BlockSpec(memory_space=pltpu.VMEM),
        ],
        out_specs=pl.BlockSpec(memory_space=pltpu.VMEM),
        scratch_shapes=[
            pltpu.VMEM((N_DEV, chunk, n), jnp.float32),
            pltpu.VMEM((N_DEV - 1, chunk, n), jnp.float32),
            pltpu.SemaphoreType.DMA((N_DEV - 1,)),
            pltpu.SemaphoreType.DMA((N_DEV - 1,)),
            pltpu.SemaphoreType.DMA((N_DEV - 1,)),
            pltpu.SemaphoreType.DMA((N_DEV - 1,)),
        ],
        compiler_params=pltpu.CompilerParams(
            vmem_limit_bytes=60 * 1024 * 1024,
        ),
    )(x, w_mat)


# device time: 406129 ns/iter; 1.2712x vs baseline; 1.2712x over previous
import jax
import jax.numpy as jnp
from jax import lax
from jax.experimental import pallas as pl
from jax.experimental.pallas import tpu as pltpu

N_DEV = 32
S = 2


def _gelu(y):
    c = 0.7978845608028654
    return 0.5 * y * (1.0 + jnp.tanh(c * (y + 0.044715 * y * y * y)))


def kernel(x, w_mat):
    m, k_loc = x.shape
    _, n = w_mat.shape
    chunk = m // N_DEV
    half = n // 2
    nsub = half // S

    def body(x_ref, w_ref, out_ref,
             rs0, rs1, ag0, ag1,
             rss0, rsr0, rss1, rsr1,
             ags0, agr0, ags1, agr1):
        d = lax.axis_index("i")

        RS = (rs0, rs1)
        AG = (ag0, ag1)
        RSS = (rss0, rss1)
        RSR = (rsr0, rsr1)
        AGS = (ags0, ags1)
        AGR = (agr0, agr1)
        SIG = (1, -1)

        def cmod(const):
            return lax.rem(d + const, N_DEV)

        def cols(di, s):
            return pl.ds(di * half + s * nsub, nsub)

        def pchunk(c, di, s):
            return jnp.dot(
                x_ref[pl.ds(c * chunk, chunk), :],
                w_ref[:, cols(di, s)],
                preferred_element_type=jnp.float32,
            )

        def rs_rdma(di, h, s):
            src = RS[di].at[N_DEV - 1, s] if h == 0 else RS[di].at[h - 1, s]
            return pltpu.make_async_remote_copy(
                src_ref=src,
                dst_ref=RS[di].at[h, s],
                send_sem=RSS[di].at[h, s],
                recv_sem=RSR[di].at[h, s],
                device_id=(cmod(N_DEV + SIG[di]),),
                device_id_type=pl.DeviceIdType.MESH,
            )

        def ag_rdma(di, g, s):
            src = RS[di].at[N_DEV - 2, s] if g == 0 else AG[di].at[g - 1, s]
            return pltpu.make_async_remote_copy(
                src_ref=src,
                dst_ref=AG[di].at[g, s],
                send_sem=AGS[di].at[g, s],
                recv_sem=AGR[di].at[g, s],
                device_id=(cmod(N_DEV + SIG[di]),),
                device_id_type=pl.DeviceIdType.MESH,
            )

        for di in (0, 1):
            for s in range(S):
                RS[di][N_DEV - 1, s] = pchunk(
                    cmod(2 * N_DEV - SIG[di]), di, s)
        for s in range(S):
            for di in (0, 1):
                rs_rdma(di, 0, s).start()

        for h in range(N_DEV - 1):
            for s in range(S):
                for di in (0, 1):
                    rs_rdma(di, h, s).wait_recv()
                    c_r = cmod(2 * N_DEV - SIG[di] * (2 + h))
                    RS[di][h, s] = RS[di][h, s] + pchunk(c_r, di, s)
                    if h < N_DEV - 2:
                        rs_rdma(di, h + 1, s).start()

        for s in range(S):
            for di in (0, 1):
                out_ref[pl.ds(d * chunk, chunk), cols(di, s)] = (
                    RS[di][N_DEV - 2, s])

        for s in range(S):
            for di in (0, 1):
                ag_rdma(di, 0, s).start()
        for g in range(N_DEV - 1):
            for s in range(S):
                for di in (0, 1):
                    ag_rdma(di, g, s).wait_recv()
                    if g < N_DEV - 2:
                        ag_rdma(di, g + 1, s).start()
                    c_r = cmod(2 * N_DEV - SIG[di] * (g + 1))
                    out_ref[pl.ds(c_r * chunk, chunk), cols(di, s)] = (
                        AG[di][g, s])

        for c in range(N_DEV):
            sl = pl.ds(c * chunk, chunk)
            out_ref[sl, :] = _gelu(out_ref[sl, :])

        for h in range(N_DEV - 1):
            for s in range(S):
                for di in (0, 1):
                    rs_rdma(di, h, s).wait_send()
                    ag_rdma(di, h, s).wait_send()

    dma = pltpu.SemaphoreType.DMA((N_DEV - 1, S))
    return pl.pallas_call(
        body,
        out_shape=jax.ShapeDtypeStruct((m, n), jnp.float32),
        in_specs=[
            pl.BlockSpec(memory_space=pltpu.VMEM),
            pl.BlockSpec(memory_space=pltpu.VMEM),
        ],
        out_specs=pl.BlockSpec(memory_space=pltpu.VMEM),
        scratch_shapes=[
            pltpu.VMEM((N_DEV, S, chunk, nsub), jnp.float32),
            pltpu.VMEM((N_DEV, S, chunk, nsub), jnp.float32),
            pltpu.VMEM((N_DEV - 1, S, chunk, nsub), jnp.float32),
            pltpu.VMEM((N_DEV - 1, S, chunk, nsub), jnp.float32),
            dma, dma, dma, dma,
            dma, dma, dma, dma,
        ],
        compiler_params=pltpu.CompilerParams(
            vmem_limit_bytes=60 * 1024 * 1024,
        ),
    )(x, w_mat)


# device time: 405141 ns/iter; 1.2743x vs baseline; 1.0024x over previous
import jax
import jax.numpy as jnp
from jax import lax
from jax.experimental import pallas as pl
from jax.experimental.pallas import tpu as pltpu

N_DEV = 32
S = 4


def _gelu(y):
    c = 0.7978845608028654
    return 0.5 * y * (1.0 + jnp.tanh(c * (y + 0.044715 * y * y * y)))


def kernel(x, w_mat):
    m, k_loc = x.shape
    _, n = w_mat.shape
    chunk = m // N_DEV
    half = n // 2
    nsub = half // S

    def body(x_ref, w_ref, out_ref, part,
             rs0, rs1,
             rss0, rsr0, rss1, rsr1,
             ags0, agr0, ags1, agr1):
        d = lax.axis_index("i")

        RS = (rs0, rs1)
        RSS = (rss0, rss1)
        RSR = (rsr0, rsr1)
        AGS = (ags0, ags1)
        AGR = (agr0, agr1)
        SIG = (1, -1)

        def cmod(const):
            return lax.rem(d + const, N_DEV)

        def rows(c):
            return pl.ds(c * chunk, chunk)

        def cols(di, s):
            return pl.ds(di * half + s * nsub, nsub)

        for c in range(N_DEV):
            part[rows(c), :] = jnp.dot(
                x_ref[rows(c), :], w_ref[:, :],
                preferred_element_type=jnp.float32,
            )

        def rs_rdma(di, h, s):
            if h == 0:
                src = part.at[rows(cmod(2 * N_DEV - SIG[di])), cols(di, s)]
            else:
                src = RS[di].at[h - 1, s]
            return pltpu.make_async_remote_copy(
                src_ref=src,
                dst_ref=RS[di].at[h, s],
                send_sem=RSS[di].at[h, s],
                recv_sem=RSR[di].at[h, s],
                device_id=(cmod(N_DEV + SIG[di]),),
                device_id_type=pl.DeviceIdType.MESH,
            )

        def ag_rdma(di, g, s):
            if g == 0:
                src = RS[di].at[N_DEV - 2, s]
            else:
                src = out_ref.at[rows(cmod(2 * N_DEV - SIG[di] * g)),
                                 cols(di, s)]
            return pltpu.make_async_remote_copy(
                src_ref=src,
                dst_ref=out_ref.at[rows(cmod(2 * N_DEV - SIG[di] * g)),
                                   cols(di, s)],
                send_sem=AGS[di].at[g, s],
                recv_sem=AGR[di].at[g, s],
                device_id=(cmod(N_DEV + SIG[di]),),
                device_id_type=pl.DeviceIdType.MESH,
            )

        def ag_wait(di, g, s):
            c_r = cmod(2 * N_DEV - SIG[di] * (g + 1))
            return pltpu.make_async_remote_copy(
                src_ref=RS[di].at[N_DEV - 2, s],
                dst_ref=out_ref.at[rows(c_r), cols(di, s)],
                send_sem=AGS[di].at[g, s],
                recv_sem=AGR[di].at[g, s],
                device_id=(cmod(N_DEV + SIG[di]),),
                device_id_type=pl.DeviceIdType.MESH,
            )

        for s in range(S):
            for di in (0, 1):
                rs_rdma(di, 0, s).start()

        for h in range(N_DEV - 1):
            for s in range(S):
                for di in (0, 1):
                    rs_rdma(di, h, s).wait_recv()
                    c_r = cmod(2 * N_DEV - SIG[di] * (2 + h))
                    RS[di][h, s] = RS[di][h, s] + part[rows(c_r), cols(di, s)]
                    if h < N_DEV - 2:
                        rs_rdma(di, h + 1, s).start()

        for s in range(S):
            for di in (0, 1):
                out_ref[rows(d), cols(di, s)] = RS[di][N_DEV - 2, s]
                ag_rdma(di, 0, s).start()

        for g in range(N_DEV - 1):
            for s in range(S):
                for di in (0, 1):
                    ag_wait(di, g, s).wait_recv()
                    if g < N_DEV - 2:
                        ag_rdma(di, g + 1, s).start()

        for h in range(N_DEV - 1):
            for s in range(S):
                for di in (0, 1):
                    rs_rdma(di, h, s).wait_send()
                    ag_rdma(di, h, s).wait_send()

        for c in range(N_DEV):
            out_ref[rows(c), :] = _gelu(out_ref[rows(c), :])

    dma = pltpu.SemaphoreType.DMA((N_DEV - 1, S))
    return pl.pallas_call(
        body,
        out_shape=jax.ShapeDtypeStruct((m, n), jnp.float32),
        in_specs=[
            pl.BlockSpec(memory_space=pltpu.VMEM),
            pl.BlockSpec(memory_space=pltpu.VMEM),
        ],
        out_specs=pl.BlockSpec(memory_space=pltpu.VMEM),
        scratch_shapes=[
            pltpu.VMEM((m, n), jnp.float32),
            pltpu.VMEM((N_DEV - 1, S, chunk, nsub), jnp.float32),
            pltpu.VMEM((N_DEV - 1, S, chunk, nsub), jnp.float32),
            dma, dma, dma, dma,
            dma, dma, dma, dma,
        ],
        compiler_params=pltpu.CompilerParams(
            vmem_limit_bytes=60 * 1024 * 1024,
        ),
    )(x, w_mat)


# device time: 228601 ns/iter; 2.2585x vs baseline; 1.7723x over previous
import os

import jax
import jax.numpy as jnp
from jax import lax
from jax.experimental import pallas as pl
from jax.experimental.pallas import tpu as pltpu

N_DEV = 32
S = 4
_ABL = os.environ.get("KABL", "full")

_TABLES = None


def _ring_tables():
    global _TABLES
    if _TABLES is not None:
        return _TABLES
    by_coords = {}
    for dv in jax.devices():
        coc = getattr(dv, "core_on_chip", None)
        if coc is not None and coc != 1:
            continue
        by_coords[tuple(getattr(dv, "coords", (dv.id,)))] = dv
    all_coords = sorted(by_coords)
    logical = []
    zs = sorted({c[2] for c in all_coords})
    for z in zs:
        plane = sorted(c for c in all_coords if c[2] == z)
        ys = sorted({c[1] for c in plane})
        for yi, y in enumerate(ys):
            row = sorted((c for c in plane if c[1] == y),
                         reverse=bool(yi % 2))
            logical.extend(row)
    logical = logical[:N_DEV]
    lidx = {c: i for i, c in enumerate(logical)}
    xs = sorted({c[0] for c in logical})
    ys = sorted({c[1] for c in logical})
    zs = sorted({c[2] for c in logical})
    if (len(xs), len(ys), len(zs)) == (2, 4, 4) and len(logical) == N_DEV:
        B = [(0, 0), (0, 1), (0, 2), (0, 3), (1, 3), (1, 2), (1, 1),
             (2, 1), (2, 2), (2, 3), (3, 3), (3, 2), (3, 1), (3, 0),
             (2, 0), (1, 0)]
        ring = ([(xs[0], ys[y], zs[z]) for (y, z) in B]
                + [(xs[1], ys[y], zs[z]) for (y, z) in reversed(B)])
        perm = [lidx[c] for c in ring]
    else:
        perm = list(range(len(logical)))
    inv = [0] * len(perm)
    for p, l in enumerate(perm):
        inv[l] = p
    _TABLES = (perm, inv)
    return _TABLES


def _gelu(y):
    c = 0.7978845608028654
    return 0.5 * y * (1.0 + jnp.tanh(c * (y + 0.044715 * y * y * y)))


def kernel(x, w_mat):
    m, k_loc = x.shape
    _, n = w_mat.shape
    chunk = m // N_DEV
    half = n // 2
    nsub = half // S

    perm_l, inv_l = _ring_tables()
    perm_arr = jnp.asarray(perm_l, jnp.int32)
    inv_arr = jnp.asarray(inv_l, jnp.int32)

    def body(x_ref, w_ref, perm_ref, inv_ref, out_ref, part,
             rs0, rs1,
             rss0, rsr0, rss1, rsr1,
             ags0, agr0, ags1, agr1):
        d = lax.axis_index("i")
        p = inv_ref[d]

        RS = (rs0, rs1)
        RSS = (rss0, rss1)
        RSR = (rsr0, rsr1)
        AGS = (ags0, ags1)
        AGR = (agr0, agr1)
        SIG = (1, -1)
        NBR = tuple(
            perm_ref[lax.rem(p + (N_DEV + sg), N_DEV)] for sg in SIG)

        def pmod(const):
            return lax.rem(p + const, N_DEV)

        def rows(c):
            return pl.ds(c * chunk, chunk)

        def cols(di, s):
            return pl.ds(di * half + s * nsub, nsub)

        for c in range(N_DEV):
            part[rows(c), :] = jnp.dot(
                x_ref[rows(c), :], w_ref[:, :],
                preferred_element_type=jnp.float32,
            )

        def rs_rdma(di, h, s):
            if h == 0:
                src = part.at[rows(pmod(2 * N_DEV - SIG[di])), cols(di, s)]
            else:
                src = RS[di].at[h - 1, s]
            return pltpu.make_async_remote_copy(
                src_ref=src,
                dst_ref=RS[di].at[h, s],
                send_sem=RSS[di].at[h, s],
                recv_sem=RSR[di].at[h, s],
                device_id=(NBR[di],),
                device_id_type=pl.DeviceIdType.MESH,
            )

        def ag_rdma(di, g, s):
            if g == 0:
                src = RS[di].at[N_DEV - 2, s]
            else:
                src = out_ref.at[rows(pmod(2 * N_DEV - SIG[di] * g)),
                                 cols(di, s)]
            return pltpu.make_async_remote_copy(
                src_ref=src,
                dst_ref=out_ref.at[rows(pmod(2 * N_DEV - SIG[di] * g)),
                                   cols(di, s)],
                send_sem=AGS[di].at[g, s],
                recv_sem=AGR[di].at[g, s],
                device_id=(NBR[di],),
                device_id_type=pl.DeviceIdType.MESH,
            )

        def ag_wait(di, g, s):
            c_r = pmod(2 * N_DEV - SIG[di] * (g + 1))
            return pltpu.make_async_remote_copy(
                src_ref=RS[di].at[N_DEV - 2, s],
                dst_ref=out_ref.at[rows(c_r), cols(di, s)],
                send_sem=AGS[di].at[g, s],
                recv_sem=AGR[di].at[g, s],
                device_id=(NBR[di],),
                device_id_type=pl.DeviceIdType.MESH,
            )

        if _ABL in ("full", "rs_only"):
            for s in range(S):
                for di in (0, 1):
                    rs_rdma(di, 0, s).start()

            for h in range(N_DEV - 1):
                for s in range(S):
                    for di in (0, 1):
                        rs_rdma(di, h, s).wait_recv()
                        c_r = pmod(2 * N_DEV - SIG[di] * (2 + h))
                        RS[di][h, s] = (
                            RS[di][h, s] + part[rows(c_r), cols(di, s)])
                        if h < N_DEV - 2:
                            rs_rdma(di, h + 1, s).start()

        if _ABL in ("full", "ag_only"):
            for s in range(S):
                for di in (0, 1):
                    out_ref[rows(p), cols(di, s)] = RS[di][N_DEV - 2, s]
                    ag_rdma(di, 0, s).start()

            for g in range(N_DEV - 1):
                for s in range(S):
                    for di in (0, 1):
                        ag_wait(di, g, s).wait_recv()
                        if g < N_DEV - 2:
                            ag_rdma(di, g + 1, s).start()

        for h in range(N_DEV - 1):
            for s in range(S):
                for di in (0, 1):
                    if _ABL in ("full", "rs_only"):
                        rs_rdma(di, h, s).wait_send()
                    if _ABL in ("full", "ag_only"):
                        ag_rdma(di, h, s).wait_send()

        for c in range(N_DEV):
            out_ref[rows(c), :] = _gelu(out_ref[rows(c), :])

    dma = pltpu.SemaphoreType.DMA((N_DEV - 1, S))
    return pl.pallas_call(
        body,
        out_shape=jax.ShapeDtypeStruct((m, n), jnp.float32),
        in_specs=[
            pl.BlockSpec(memory_space=pltpu.VMEM),
            pl.BlockSpec(memory_space=pltpu.VMEM),
            pl.BlockSpec(memory_space=pltpu.SMEM),
            pl.BlockSpec(memory_space=pltpu.SMEM),
        ],
        out_specs=pl.BlockSpec(memory_space=pltpu.VMEM),
        scratch_shapes=[
            pltpu.VMEM((m, n), jnp.float32),
            pltpu.VMEM((N_DEV - 1, S, chunk, nsub), jnp.float32),
            pltpu.VMEM((N_DEV - 1, S, chunk, nsub), jnp.float32),
            dma, dma, dma, dma,
            dma, dma, dma, dma,
        ],
        compiler_params=pltpu.CompilerParams(
            vmem_limit_bytes=60 * 1024 * 1024,
        ),
    )(x, w_mat, perm_arr, inv_arr)


# device time: 214008 ns/iter; 2.4125x vs baseline; 1.0682x over previous
import jax
import jax.numpy as jnp
from jax import lax
from jax.experimental import pallas as pl
from jax.experimental.pallas import tpu as pltpu

N_DEV = 32
S = 4

_TABLES = None


def _ring_tables():
    global _TABLES
    if _TABLES is not None:
        return _TABLES
    by_coords = {}
    for dv in jax.devices():
        coc = getattr(dv, "core_on_chip", None)
        if coc is not None and coc != 1:
            continue
        by_coords[tuple(getattr(dv, "coords", (dv.id,)))] = dv
    all_coords = sorted(by_coords)
    logical = []
    zs = sorted({c[2] for c in all_coords})
    for z in zs:
        plane = sorted(c for c in all_coords if c[2] == z)
        ys = sorted({c[1] for c in plane})
        for yi, y in enumerate(ys):
            row = sorted((c for c in plane if c[1] == y),
                         reverse=bool(yi % 2))
            logical.extend(row)
    logical = logical[:N_DEV]
    lidx = {c: i for i, c in enumerate(logical)}
    xs = sorted({c[0] for c in logical})
    ys = sorted({c[1] for c in logical})
    zs = sorted({c[2] for c in logical})
    if (len(xs), len(ys), len(zs)) == (2, 4, 4) and len(logical) == N_DEV:
        B = [(0, 0), (0, 1), (0, 2), (0, 3), (1, 3), (1, 2), (1, 1),
             (2, 1), (2, 2), (2, 3), (3, 3), (3, 2), (3, 1), (3, 0),
             (2, 0), (1, 0)]
        ring = ([(xs[0], ys[y], zs[z]) for (y, z) in B]
                + [(xs[1], ys[y], zs[z]) for (y, z) in reversed(B)])
        perm = [lidx[c] for c in ring]
    else:
        perm = list(range(len(logical)))
    inv = [0] * len(perm)
    for p, l in enumerate(perm):
        inv[l] = p
    _TABLES = (perm, inv)
    return _TABLES


def _gelu(y):
    c = 0.7978845608028654
    return 0.5 * y * (1.0 + jnp.tanh(c * (y + 0.044715 * y * y * y)))


def kernel(x, w_mat):
    m, k_loc = x.shape
    _, n = w_mat.shape
    chunk = m // N_DEV
    half = n // 2
    nsub = half // S

    perm_l, inv_l = _ring_tables()
    perm_arr = jnp.asarray(perm_l, jnp.int32)
    inv_arr = jnp.asarray(inv_l, jnp.int32)

    def body(x_ref, w_ref, perm_ref, inv_ref, out_ref, part,
             rs0, rs1,
             rss0, rsr0, rss1, rsr1,
             ags0, agr0, ags1, agr1):
        d = lax.axis_index("i")
        p = inv_ref[d]

        RS = (rs0, rs1)
        RSS = (rss0, rss1)
        RSR = (rsr0, rsr1)
        AGS = (ags0, ags1)
        AGR = (agr0, agr1)
        SIG = (1, -1)
        NBR = tuple(
            perm_ref[lax.rem(p + (N_DEV + sg), N_DEV)] for sg in SIG)

        def pmod(const):
            return lax.rem(p + const, N_DEV)

        def rows(c):
            return pl.ds(c * chunk, chunk)

        def halfcols(di):
            return pl.ds(di * half, half)

        def cols(di, s):
            return pl.ds(di * half + s * nsub, nsub)

        def piece(di, k):
            r = rows(pmod(2 * N_DEV - SIG[di] * k))
            part[r, halfcols(di)] = jnp.dot(
                x_ref[r, :], w_ref[:, halfcols(di)],
                preferred_element_type=jnp.float32,
            )

        def rs_rdma(di, h, s):
            if h == 0:
                src = part.at[rows(pmod(2 * N_DEV - SIG[di])), cols(di, s)]
            else:
                src = RS[di].at[h - 1, s]
            return pltpu.make_async_remote_copy(
                src_ref=src,
                dst_ref=RS[di].at[h, s],
                send_sem=RSS[di].at[h, s],
                recv_sem=RSR[di].at[h, s],
                device_id=(NBR[di],),
                device_id_type=pl.DeviceIdType.MESH,
            )

        def ag_rdma(di, g, s):
            if g == 0:
                src = RS[di].at[N_DEV - 2, s]
            else:
                src = out_ref.at[rows(pmod(2 * N_DEV - SIG[di] * g)),
                                 cols(di, s)]
            return pltpu.make_async_remote_copy(
                src_ref=src,
                dst_ref=out_ref.at[rows(pmod(2 * N_DEV - SIG[di] * g)),
                                   cols(di, s)],
                send_sem=AGS[di].at[g, s],
                recv_sem=AGR[di].at[g, s],
                device_id=(NBR[di],),
                device_id_type=pl.DeviceIdType.MESH,
            )

        def ag_wait(di, g, s):
            c_r = pmod(2 * N_DEV - SIG[di] * (g + 1))
            return pltpu.make_async_remote_copy(
                src_ref=RS[di].at[N_DEV - 2, s],
                dst_ref=out_ref.at[rows(c_r), cols(di, s)],
                send_sem=AGS[di].at[g, s],
                recv_sem=AGR[di].at[g, s],
                device_id=(NBR[di],),
                device_id_type=pl.DeviceIdType.MESH,
            )

        for di in (0, 1):
            for k in (1, 2, 3):
                piece(di, k)

        barrier_sem = pltpu.get_barrier_semaphore()
        for di in (0, 1):
            pl.semaphore_signal(
                barrier_sem, inc=1,
                device_id=(NBR[di],),
                device_id_type=pl.DeviceIdType.MESH,
            )
        pl.semaphore_wait(barrier_sem, 2)

        for s in range(S):
            for di in (0, 1):
                rs_rdma(di, 0, s).start()

        for h in range(N_DEV - 1):
            for s in range(S):
                for di in (0, 1):
                    rs_rdma(di, h, s).wait_recv()
                    c_r = pmod(2 * N_DEV - SIG[di] * (2 + h))
                    RS[di][h, s] = (
                        RS[di][h, s] + part[rows(c_r), cols(di, s)])
                    if h < N_DEV - 2:
                        rs_rdma(di, h + 1, s).start()
                if s == 0 and h <= N_DEV - 4:
                    for di in (0, 1):
                        piece(di, h + 4)

        for s in range(S):
            for di in (0, 1):
                out_ref[rows(p), cols(di, s)] = RS[di][N_DEV - 2, s]
                ag_rdma(di, 0, s).start()
        out_ref[rows(p), :] = _gelu(out_ref[rows(p), :])

        for g in range(N_DEV - 1):
            for s in range(S):
                for di in (0, 1):
                    ag_wait(di, g, s).wait_recv()
                    if g < N_DEV - 2:
                        ag_rdma(di, g + 1, s).start()
            if g >= 2:
                for di in (0, 1):
                    for s in range(S):
                        ag_rdma(di, g - 1, s).wait_send()
                    r = rows(pmod(2 * N_DEV - SIG[di] * (g - 1)))
                    out_ref[r, halfcols(di)] = _gelu(out_ref[r, halfcols(di)])

        for di in (0, 1):
            for s in range(S):
                ag_rdma(di, N_DEV - 2, s).wait_send()
            r = rows(pmod(2 * N_DEV - SIG[di] * (N_DEV - 2)))
            out_ref[r, halfcols(di)] = _gelu(out_ref[r, halfcols(di)])
            r = rows(pmod(2 * N_DEV - SIG[di] * (N_DEV - 1)))
            out_ref[r, halfcols(di)] = _gelu(out_ref[r, halfcols(di)])

        for h in range(N_DEV - 1):
            for s in range(S):
                for di in (0, 1):
                    rs_rdma(di, h, s).wait_send()
        for s in range(S):
            for di in (0, 1):
                ag_rdma(di, 0, s).wait_send()

    dma = pltpu.SemaphoreType.DMA((N_DEV - 1, S))
    return pl.pallas_call(
        body,
        out_shape=jax.ShapeDtypeStruct((m, n), jnp.float32),
        in_specs=[
            pl.BlockSpec(memory_space=pltpu.VMEM),
            pl.BlockSpec(memory_space=pltpu.VMEM),
            pl.BlockSpec(memory_space=pltpu.SMEM),
            pl.BlockSpec(memory_space=pltpu.SMEM),
        ],
        out_specs=pl.BlockSpec(memory_space=pltpu.VMEM),
        scratch_shapes=[
            pltpu.VMEM((m, n), jnp.float32),
            pltpu.VMEM((N_DEV - 1, S, chunk, nsub), jnp.float32),
            pltpu.VMEM((N_DEV - 1, S, chunk, nsub), jnp.float32),
            dma, dma, dma, dma,
            dma, dma, dma, dma,
        ],
        compiler_params=pltpu.CompilerParams(
            vmem_limit_bytes=60 * 1024 * 1024,
            collective_id=0,
        ),
    )(x, w_mat, perm_arr, inv_arr)
